# baseline (speedup 1.0000x reference)
import functools

import jax
import jax.numpy as jnp
from jax import lax
from jax.experimental import pallas as pl
from jax.experimental.pallas import tpu as pltpu
from jax.experimental.pallas import tpu_sc as plsc

_NC = 2
_NS = 16
_NW = _NC * _NS
_L = 16
_CH = 128
_Q = 32


def _sc_call(n_nodes, n_edges, d_feat, d_out):
    n_cols = n_edges // _CH
    cols_low = n_cols // _NW
    n_extra = n_cols % _NW
    n_iters = cols_low + (1 if n_extra else 0)
    n_iters += n_iters % 2

    mesh = plsc.VectorSubcoreMesh(core_axis_name="c", subcore_axis_name="s")

    @functools.partial(
        pl.kernel,
        out_type=jax.ShapeDtypeStruct((d_out, n_edges), jnp.float32),
        mesh=mesh,
        scratch_types=[
            pltpu.VMEM_SHARED((n_nodes, d_feat), jnp.float32),
            pltpu.VMEM((_CH,), jnp.int32),
            pltpu.VMEM((_CH,), jnp.int32),
            pltpu.VMEM((_CH,), jnp.int32),
            pltpu.VMEM((_CH,), jnp.int32),
            pltpu.VMEM((3, _CH), jnp.int32),
            pltpu.VMEM((3, _CH), jnp.int32),
            pltpu.VMEM((3, _CH), jnp.float32),
            pltpu.VMEM((_Q, d_feat), jnp.float32),
            pltpu.VMEM((_Q, d_feat), jnp.float32),
            pltpu.VMEM((d_out, _CH), jnp.float32),
            pltpu.VMEM((d_out, _CH), jnp.float32),
            pltpu.SemaphoreType.DMA,
            pltpu.SemaphoreType.DMA,
            pltpu.SemaphoreType.DMA,
            pltpu.SemaphoreType.DMA,
            pltpu.SemaphoreType.DMA,
            pltpu.SemaphoreType.DMA,
            pltpu.SemaphoreType.DMA,
            pltpu.SemaphoreType.DMA,
            pltpu.SemaphoreType.DMA,
            pltpu.SemaphoreType.DMA,
        ],
        compiler_params=pltpu.CompilerParams(needs_layout_passes=False),
    )
    def sc_kernel(feat_hbm, posf_hbm, src_hbm, dst_hbm, out_hbm,
                  feat_sh, sv0, sv1, dv0, dv1, is3, id3, ptmp,
                  fq0, fq1, bf0, bf1,
                  ssem0, ssem1, dsem0, dsem1, qsem0, qsem1, psem,
                  osem0, osem1, stsem):
        wid = lax.axis_index("s") * _NC + lax.axis_index("c")
        n_mine = cols_low + jnp.where(wid < n_extra, 1, 0)
        iota = jnp.arange(_L, dtype=jnp.int32)

        sv = (sv0, sv1)
        dv = (dv0, dv1)
        fq = (fq0, fq1)
        bf = (bf0, bf1)
        ssem = (ssem0, ssem1)
        dsem = (dsem0, dsem1)
        qsem = (qsem0, qsem1)
        osem = (osem0, osem1)

        def col_of(g):
            return wid + _NW * jnp.minimum(g, n_mine - 1)

        def issue_idx(g, b):
            base = col_of(g) * _CH
            pltpu.async_copy(src_hbm.at[pl.ds(base, _CH)], sv[b], ssem[b])
            pltpu.async_copy(dst_hbm.at[pl.ds(base, _CH)], dv[b], dsem[b])

        def wait_idx(b):
            pltpu.make_async_copy(src_hbm.at[pl.ds(0, _CH)], sv[b], ssem[b]).wait()
            pltpu.make_async_copy(dst_hbm.at[pl.ds(0, _CH)], dv[b], dsem[b]).wait()

        def issue_q(q, b):
            pltpu.async_copy(feat_sh.at[sv[b].at[pl.ds(q * _Q, _Q)]],
                             fq[q % 2], qsem[q % 2])

        def wait_q(q, b):
            pltpu.make_async_copy(feat_sh.at[sv[b].at[pl.ds(0, _Q)]],
                                  fq[q % 2], qsem[q % 2]).wait()

        def issue_write(g, b):
            pltpu.async_copy(
                bf[b], out_hbm.at[:, pl.ds(col_of(g) * _CH, _CH)], osem[b])

        def wait_write(b):
            pltpu.make_async_copy(
                bf[b], out_hbm.at[:, pl.ds(0, _CH)], osem[b]).wait()

        diag = [jnp.bitwise_and(iota + d, _L - 1) for d in range(_L)]
        n_cb = d_feat // _L

        def do_iter(g, b, i):
            svb, dvb, bfb = sv[b], dv[b], bf[b]
            wait_idx(b)

            def bld(grp, c2):
                e16 = iota + grp * _L
                s16 = svb[pl.ds(grp * _L, _L)] * 3
                d16 = dvb[pl.ds(grp * _L, _L)] * 3
                for c in range(3):
                    cc = jnp.full((_L,), c, dtype=jnp.int32)
                    plsc.store_scatter(is3, [cc, e16], s16 + c)
                    plsc.store_scatter(id3, [cc, e16], d16 + c)
                return c2

            lax.fori_loop(0, _CH // _L, bld, 0)

            @pl.when(i >= 1)
            def _():
                wait_write(b)

            for c in range(3):
                pltpu.async_copy(posf_hbm.at[is3.at[c]], bfb.at[c], psem)
                pltpu.async_copy(posf_hbm.at[id3.at[c]], ptmp.at[c], psem)

            issue_q(0, b)
            for q in range(_CH // _Q):
                if q + 1 < _CH // _Q:
                    issue_q(q + 1, b)
                wait_q(q, b)
                fqq = fq[q % 2]

                def blk(j, c2, _q=q, _fqq=fqq):
                    rl16 = (j >> 3) * _L + iota
                    e16 = _q * _Q + rl16
                    cb = (jnp.bitwise_and(j, n_cb - 1)) * _L
                    for h in range(2):
                        srcc = [cb + diag[h * 8 + d] for d in range(8)]
                        vals = [plsc.load_gather(_fqq, [rl16, srcc[d]])
                                for d in range(8)]
                        for d in range(8):
                            plsc.store_scatter(bfb, [srcc[d] + 3, e16], vals[d])
                    return c2

                lax.fori_loop(0, (_Q // _L) * n_cb, blk, 0)

            for c in range(3):
                pltpu.make_async_copy(posf_hbm.at[is3.at[c]], bfb.at[c], psem).wait()
                pltpu.make_async_copy(posf_hbm.at[id3.at[c]], ptmp.at[c], psem).wait()
            for c in range(3):
                for grp in range(_CH // _L):
                    sl = pl.ds(grp * _L, _L)
                    bfb[c, sl] = bfb[c, sl] - ptmp[c, sl]

            issue_write(g, b)
            issue_idx(g + 2, b)

        @pl.when(lax.axis_index("s") == 0)
        def _():
            pltpu.async_copy(feat_hbm, feat_sh, stsem).wait()

        plsc.subcore_barrier()
        issue_idx(0, 0)
        issue_idx(1, 1)

        def pair(i, carry):
            g0 = 2 * i
            do_iter(g0, 0, i)
            do_iter(g0 + 1, 1, i)
            return carry

        lax.fori_loop(0, n_iters // 2, pair, 0)

        wait_idx(0)
        wait_idx(1)
        wait_write(0)
        wait_write(1)

    return sc_kernel


def kernel(pos, feat, edge_index):
    n_nodes, d_feat = feat.shape
    n_edges = edge_index.shape[1]
    d_out = d_feat + 3
    pos_flat = pos.reshape(-1)
    src = edge_index[0].astype(jnp.int32)
    dst = edge_index[1].astype(jnp.int32)
    fn = _sc_call(n_nodes, n_edges, d_feat, d_out)
    out_t = fn(feat, pos_flat, src, dst)
    return out_t.T


# speedup vs baseline: 1.1216x; 1.1216x over previous
import functools

import jax
import jax.numpy as jnp
from jax import lax
from jax.experimental import pallas as pl
from jax.experimental.pallas import tpu as pltpu
from jax.experimental.pallas import tpu_sc as plsc

_NC = 2
_NS = 16
_NW = _NC * _NS
_L = 16
_CH = 128


def _sc_call(n_nodes, n_edges, d_feat, d_out):
    n_cols = n_edges // _CH
    cols_low = n_cols // _NW
    n_extra = n_cols % _NW
    n_iters = cols_low + (1 if n_extra else 0)
    n_iters += n_iters % 2

    mesh = plsc.VectorSubcoreMesh(core_axis_name="c", subcore_axis_name="s")

    @functools.partial(
        pl.kernel,
        out_type=jax.ShapeDtypeStruct((d_out, n_edges), jnp.float32),
        mesh=mesh,
        scratch_types=[
            pltpu.VMEM((n_nodes * 3,), jnp.float32),
            pltpu.VMEM((_CH,), jnp.int32),
            pltpu.VMEM((_CH,), jnp.int32),
            pltpu.VMEM((_CH,), jnp.int32),
            pltpu.VMEM((_CH,), jnp.int32),
            pltpu.VMEM((_CH, d_feat), jnp.float32),
            pltpu.VMEM((_CH, d_feat), jnp.float32),
            pltpu.VMEM((d_out, _CH), jnp.float32),
            pltpu.VMEM((d_out, _CH), jnp.float32),
            pltpu.SemaphoreType.DMA,
            pltpu.SemaphoreType.DMA,
            pltpu.SemaphoreType.DMA,
            pltpu.SemaphoreType.DMA,
            pltpu.SemaphoreType.DMA,
            pltpu.SemaphoreType.DMA,
            pltpu.SemaphoreType.DMA,
            pltpu.SemaphoreType.DMA,
        ],
        compiler_params=pltpu.CompilerParams(needs_layout_passes=False),
    )
    def sc_kernel(feat_hbm, posf_hbm, src_hbm, dst_hbm, out_hbm,
                  posv, sv0, sv1, dv0, dv1, fb0, fb1, bf0, bf1,
                  ssem0, ssem1, dsem0, dsem1, gsem0, gsem1, osem0, osem1):
        wid = lax.axis_index("s") * _NC + lax.axis_index("c")
        n_mine = cols_low + jnp.where(wid < n_extra, 1, 0)
        iota = jnp.arange(_L, dtype=jnp.int32)

        sv = (sv0, sv1)
        dv = (dv0, dv1)
        fb = (fb0, fb1)
        bf = (bf0, bf1)
        ssem = (ssem0, ssem1)
        dsem = (dsem0, dsem1)
        gsem = (gsem0, gsem1)
        osem = (osem0, osem1)

        def col_of(g):
            return wid + _NW * jnp.minimum(g, n_mine - 1)

        def issue_idx(g, b):
            base = col_of(g) * _CH
            pltpu.async_copy(src_hbm.at[pl.ds(base, _CH)], sv[b], ssem[b])
            pltpu.async_copy(dst_hbm.at[pl.ds(base, _CH)], dv[b], dsem[b])

        def wait_idx(b):
            pltpu.make_async_copy(src_hbm.at[pl.ds(0, _CH)], sv[b], ssem[b]).wait()
            pltpu.make_async_copy(dst_hbm.at[pl.ds(0, _CH)], dv[b], dsem[b]).wait()

        def issue_gather(b):
            pltpu.async_copy(feat_hbm.at[sv[b]], fb[b], gsem[b])

        def wait_gather(b):
            pltpu.make_async_copy(feat_hbm.at[sv[b]], fb[b], gsem[b]).wait()

        def issue_write(g, b):
            pltpu.async_copy(
                bf[b], out_hbm.at[:, pl.ds(col_of(g) * _CH, _CH)], osem[b])

        def wait_write(b):
            pltpu.make_async_copy(
                bf[b], out_hbm.at[:, pl.ds(0, _CH)], osem[b]).wait()

        def compute(b):
            svb, dvb, fbb, bfb = sv[b], dv[b], fb[b], bf[b]

            def rel_grp(i, c2):
                s16 = svb[pl.ds(i * _L, _L)]
                d16 = dvb[pl.ds(i * _L, _L)]
                e16 = iota + i * _L
                for c in range(3):
                    cc = jnp.full((_L,), c, dtype=jnp.int32)
                    ps = plsc.load_gather(posv, [s16 * 3 + c])
                    pd = plsc.load_gather(posv, [d16 * 3 + c])
                    plsc.store_scatter(bfb, [cc, e16], ps - pd)
                return c2

            lax.fori_loop(0, _CH // _L, rel_grp, 0)

            diag = [jnp.bitwise_and(iota + d, _L - 1) for d in range(_L)]
            n_cb = d_feat // _L

            def blk(i, c2):
                r16 = (i >> 3) * _L + iota
                cb = (jnp.bitwise_and(i, n_cb - 1)) * _L
                for h in range(2):
                    srcc = [cb + diag[h * 8 + d] for d in range(8)]
                    vals = [plsc.load_gather(fbb, [r16, srcc[d]])
                            for d in range(8)]
                    for d in range(8):
                        plsc.store_scatter(bfb, [srcc[d] + 3, r16], vals[d])
                return c2

            lax.fori_loop(0, (_CH // _L) * n_cb, blk, 0)

        def do_iter(g, b, i):
            wait_gather(b)
            wait_idx(1 - b)
            issue_gather(1 - b)

            @pl.when(i >= 1)
            def _():
                wait_write(b)

            compute(b)
            issue_write(g, b)
            issue_idx(g + 2, b)

        pltpu.sync_copy(posf_hbm, posv)
        issue_idx(0, 0)
        issue_idx(1, 1)
        wait_idx(0)
        issue_gather(0)

        def pair(i, carry):
            g0 = 2 * i
            do_iter(g0, 0, i)
            do_iter(g0 + 1, 1, i)
            return carry

        lax.fori_loop(0, n_iters // 2, pair, 0)

        wait_gather(0)
        wait_idx(1)
        wait_write(0)
        wait_write(1)

    return sc_kernel


def kernel(pos, feat, edge_index):
    n_nodes, d_feat = feat.shape
    n_edges = edge_index.shape[1]
    d_out = d_feat + 3
    pos_flat = pos.reshape(-1)
    src = edge_index[0].astype(jnp.int32)
    dst = edge_index[1].astype(jnp.int32)
    fn = _sc_call(n_nodes, n_edges, d_feat, d_out)
    out_t = fn(feat, pos_flat, src, dst)
    return out_t.T
